# baseline (speedup 1.0000x reference)
import jax
import jax.numpy as jnp
from jax.experimental import pallas as pl
from jax.experimental.pallas import tpu as pltpu

B, H, S, D = 8, 16, 4096, 128
Q = 16
BH = B * H
NCH = 8


def _body(pos_ref, kval_hbm, vval_hbm, kc_hbm, vc_hbm, ko_hbm, vo_hbm,
          kbuf, vbuf, copy_sem, val_sem, row_sem):
    rows = BH // NCH
    for c in range(NCH):
        sl = pl.ds(c * rows, rows)
        pltpu.make_async_copy(kc_hbm.at[sl], ko_hbm.at[sl], copy_sem.at[c]).start()
        pltpu.make_async_copy(vc_hbm.at[sl], vo_hbm.at[sl], copy_sem.at[NCH + c]).start()

    pltpu.make_async_copy(kval_hbm, kbuf, val_sem).start()
    pltpu.make_async_copy(vval_hbm, vbuf, val_sem).start()
    pltpu.make_async_copy(kval_hbm, kbuf, val_sem).wait()
    pltpu.make_async_copy(vval_hbm, vbuf, val_sem).wait()

    for q in range(Q):
        m = q
        for r in range(q + 1, Q):
            m = jnp.where(pos_ref[r] == pos_ref[q], r, m)

        @pl.when(m != q)
        def _():
            kbuf[:, pl.ds(q, 1), :] = kbuf[:, pl.ds(m, 1), :]
            vbuf[:, pl.ds(q, 1), :] = vbuf[:, pl.ds(m, 1), :]

    for c in range(NCH):
        pltpu.make_async_copy(kc_hbm.at[pl.ds(c * rows, rows)],
                              ko_hbm.at[pl.ds(c * rows, rows)], copy_sem.at[c]).wait()
        pltpu.make_async_copy(vc_hbm.at[pl.ds(c * rows, rows)],
                              vo_hbm.at[pl.ds(c * rows, rows)], copy_sem.at[NCH + c]).wait()

    for q in range(Q):
        p = pos_ref[q]
        pltpu.make_async_copy(kbuf.at[:, pl.ds(q, 1), :],
                              ko_hbm.at[:, pl.ds(p, 1), :], row_sem).start()
        pltpu.make_async_copy(vbuf.at[:, pl.ds(q, 1), :],
                              vo_hbm.at[:, pl.ds(p, 1), :], row_sem).start()
    for q in range(Q):
        p = pos_ref[q]
        pltpu.make_async_copy(kbuf.at[:, pl.ds(q, 1), :],
                              ko_hbm.at[:, pl.ds(p, 1), :], row_sem).wait()
        pltpu.make_async_copy(vbuf.at[:, pl.ds(q, 1), :],
                              vo_hbm.at[:, pl.ds(p, 1), :], row_sem).wait()


def kernel(input_pos, k_val, v_val, k_cache, v_cache):
    kc = k_cache.reshape(BH, S, D)
    vc = v_cache.reshape(BH, S, D)
    kv = k_val.reshape(BH, Q, D)
    vv = v_val.reshape(BH, Q, D)
    any_spec = pl.BlockSpec(memory_space=pl.ANY)
    ko, vo = pl.pallas_call(
        _body,
        in_specs=[
            pl.BlockSpec(memory_space=pltpu.SMEM),
            any_spec, any_spec, any_spec, any_spec,
        ],
        out_specs=[any_spec, any_spec],
        out_shape=[
            jax.ShapeDtypeStruct((BH, S, D), jnp.float32),
            jax.ShapeDtypeStruct((BH, S, D), jnp.float32),
        ],
        scratch_shapes=[
            pltpu.VMEM((BH, Q, D), jnp.float32),
            pltpu.VMEM((BH, Q, D), jnp.float32),
            pltpu.SemaphoreType.DMA((2 * NCH,)),
            pltpu.SemaphoreType.DMA,
            pltpu.SemaphoreType.DMA,
        ],
    )(input_pos, kv, vv, kc, vc)
    return ko.reshape(B, H, S, D), vo.reshape(B, H, S, D)


# speedup vs baseline: 48.2813x; 48.2813x over previous
import jax
import jax.numpy as jnp
from jax.experimental import pallas as pl
from jax.experimental.pallas import tpu as pltpu

B, H, S, D = 8, 16, 4096, 128
Q = 16
BH = B * H


def _body(pos_ref, kval_hbm, vval_hbm, kc_hbm, vc_hbm, ko_hbm, vo_hbm,
          kbuf, vbuf, val_sem, row_sem):
    pltpu.make_async_copy(kval_hbm, kbuf, val_sem).start()
    pltpu.make_async_copy(vval_hbm, vbuf, val_sem).start()
    pltpu.make_async_copy(kval_hbm, kbuf, val_sem).wait()
    pltpu.make_async_copy(vval_hbm, vbuf, val_sem).wait()

    for q in range(Q):
        m = q
        for r in range(q + 1, Q):
            m = jnp.where(pos_ref[r] == pos_ref[q], r, m)

        @pl.when(m != q)
        def _():
            kbuf[:, pl.ds(q, 1), :] = kbuf[:, pl.ds(m, 1), :]
            vbuf[:, pl.ds(q, 1), :] = vbuf[:, pl.ds(m, 1), :]

    for q in range(Q):
        p = pos_ref[q]
        pltpu.make_async_copy(kbuf.at[:, pl.ds(q, 1), :],
                              ko_hbm.at[:, pl.ds(p, 1), :], row_sem).start()
        pltpu.make_async_copy(vbuf.at[:, pl.ds(q, 1), :],
                              vo_hbm.at[:, pl.ds(p, 1), :], row_sem).start()
    for q in range(Q):
        p = pos_ref[q]
        pltpu.make_async_copy(kbuf.at[:, pl.ds(q, 1), :],
                              ko_hbm.at[:, pl.ds(p, 1), :], row_sem).wait()
        pltpu.make_async_copy(vbuf.at[:, pl.ds(q, 1), :],
                              vo_hbm.at[:, pl.ds(p, 1), :], row_sem).wait()


def kernel(input_pos, k_val, v_val, k_cache, v_cache):
    kc = k_cache.reshape(BH, S, D)
    vc = v_cache.reshape(BH, S, D)
    kv = k_val.reshape(BH, Q, D)
    vv = v_val.reshape(BH, Q, D)
    any_spec = pl.BlockSpec(memory_space=pl.ANY)
    ko, vo = pl.pallas_call(
        _body,
        in_specs=[
            pl.BlockSpec(memory_space=pltpu.SMEM),
            any_spec, any_spec, any_spec, any_spec,
        ],
        out_specs=[any_spec, any_spec],
        out_shape=[
            jax.ShapeDtypeStruct((BH, S, D), jnp.float32),
            jax.ShapeDtypeStruct((BH, S, D), jnp.float32),
        ],
        input_output_aliases={3: 0, 4: 1},
        scratch_shapes=[
            pltpu.VMEM((BH, Q, D), jnp.float32),
            pltpu.VMEM((BH, Q, D), jnp.float32),
            pltpu.SemaphoreType.DMA,
            pltpu.SemaphoreType.DMA,
        ],
    )(input_pos, kv, vv, kc, vc)
    return ko.reshape(B, H, S, D), vo.reshape(B, H, S, D)
